# baseline (speedup 1.0000x reference)
import functools

import jax
import jax.numpy as jnp
import numpy as np
from jax import lax
from jax.experimental import pallas as pl
from jax.experimental.pallas import tpu as pltpu
from jax.experimental.pallas import tpu_sc as plsc

N = 32768
NT = 16
CH = N // NT
VR = CH // 16
RB = 8
B = 1 << RB
NG = B // 16
NPASS = 4
MSB = np.int32(-2147483648)


def _iota16():
    return lax.iota(jnp.int32, 16)


def _sc_body(u_hbm, out_hbm, kv, pv, kg, posv, dv, occv, lv, histv, offv,
             totv, omat, gtmp, gv_all, idxm, bw, flagmine, flagv, okv,
             sem, semp, semk,
             sk0, sp0, sk1, sp1, korig, stot, soff, sgrp, sflag):
    t = lax.axis_index("s")
    it = _iota16()
    base = t * CH
    zero16 = jnp.zeros((16,), jnp.int32)

    pltpu.sync_copy(u_hbm.at[pl.ds(base, CH)], kv)

    def key_j(j, _):
        u = kv[pl.ds(j * 16, 16)]
        s = lax.shift_right_arithmetic(u, 31)
        kv[pl.ds(j * 16, 16)] = (u ^ ~s) & (s | jnp.int32(0x7FFFFFFF))
        pv[pl.ds(j * 16, 16)] = base + j * 16 + it
        return 0

    lax.fori_loop(0, VR, key_j, 0)
    korig_in = pltpu.async_copy(kv, korig.at[pl.ds(base, CH)], semk)

    for r in range(2):
        for q in range(8):
            plsc.store_scatter(
                idxm, [zero16 + r, q * 16 + it],
                jnp.int32((r * 8 + q) * B) + t * 16 + it)

    bufs = [(sk0, sp0), (sk1, sp1)]
    pv_in = None
    for p in range(NPASS):
        shift = RB * p
        k_out, p_out = bufs[p % 2]
        kv_a = kv_b = None
        if p > 0:
            k_in, p_in = bufs[(p + 1) % 2]
            pv_in = pltpu.async_copy(p_in.at[pl.ds(base, CH)], pv, semp)
            H = CH // 2
            kv_a = pltpu.async_copy(
                k_in.at[pl.ds(base, H)], kv.at[pl.ds(0, H)], semk)
            kv_b = pltpu.async_copy(
                k_in.at[pl.ds(base + H, H)], kv.at[pl.ds(H, H)], semk)

        for g in range(NG):
            histv[pl.ds(g * 16, 16)] = zero16
        if kv_a is not None:
            kv_a.wait()

        def hist_j(j, _):
            kvec = kv[pl.ds(j * 16, 16)]
            d = lax.shift_right_logical(kvec, shift) & (B - 1)
            occ, last = plsc.scan_count(d)
            dv[pl.ds(j * 16, 16)] = d
            occv[pl.ds(j * 16, 16)] = occ
            lv[pl.ds(j * 16, 16)] = jnp.where(last, 1, 0)
            plsc.addupdate_scatter(histv, [d], occ, mask=last)
            return 0

        lax.fori_loop(0, VR // 2, hist_j, 0)
        if kv_b is not None:
            kv_b.wait()
        lax.fori_loop(VR // 2, VR, hist_j, 0)

        pltpu.sync_copy(histv, stot.at[pl.ds(t * B, B)])
        plsc.subcore_barrier()
        pltpu.sync_copy(stot, totv)

        gbase = t * 16
        rows = [totv[pl.ds(tp * B + gbase, 16)] for tp in range(NT)]
        tot_g = zero16
        for tp in range(NT):
            tot_g = tot_g + rows[tp]
        wge = plsc.cumsum(tot_g) - tot_g
        acc = zero16
        for tp in range(NT):
            omat[pl.ds(tp * 16, 16)] = wge + acc
            acc = acc + rows[tp]
        gtmp[0:16] = jnp.where(it == t, jnp.sum(tot_g), 0)
        pltpu.sync_copy(gtmp, sgrp.at[pl.ds(t * 16, 16)])
        oc0 = pltpu.async_copy(
            omat.at[pl.ds(0, 128)], soff.at[idxm.at[0]], sem)
        oc1 = pltpu.async_copy(
            omat.at[pl.ds(128, 128)], soff.at[idxm.at[1]], sem)
        oc0.wait()
        oc1.wait()
        plsc.subcore_barrier()

        pltpu.sync_copy(soff.at[pl.ds(t * B, B)], offv)
        pltpu.sync_copy(sgrp, gv_all)
        gt = gv_all[pl.ds(0, 16)]
        for tp in range(1, NT):
            gt = gt + gv_all[pl.ds(tp * 16, 16)]
        gexc = plsc.cumsum(gt) - gt
        for g in range(NG):
            bg = jnp.sum(jnp.where(it == g, gexc, 0))
            offv[pl.ds(g * 16, 16)] = offv[pl.ds(g * 16, 16)] + bg

        if pv_in is not None:
            pv_in.wait()

        def perm_j(j, _):
            d = dv[pl.ds(j * 16, 16)]
            occ = occv[pl.ds(j * 16, 16)]
            last = lv[pl.ds(j * 16, 16)] != 0
            pos = plsc.load_gather(offv, [d]) + occ - 1
            jd = lax.div(j, 8)
            jm = j - jd * 8
            plsc.store_scatter(posv, [zero16 + jd, jm * 16 + it], pos)
            plsc.addupdate_scatter(offv, [d], occ, mask=last)

            @pl.when(jm == 7)
            def _():
                pltpu.async_copy(
                    kv.at[pl.ds(jd * VR, VR)], k_out.at[posv.at[jd]], sem)
                pltpu.async_copy(
                    pv.at[pl.ds(jd * VR, VR)], p_out.at[posv.at[jd]], sem)

            return 0

        lax.fori_loop(0, VR, perm_j, 0)

        for _ in range(2 * NT):
            pltpu.make_async_copy(
                u_hbm.at[pl.ds(0, VR)], kg.at[pl.ds(0, VR)], sem).wait()
        if p == 0:
            korig_in.wait()
        plsc.subcore_barrier()

    ks, ps = bufs[(NPASS - 1) % 2]
    kv_back = pltpu.async_copy(ks.at[pl.ds(base, CH)], kv, semk)
    pltpu.sync_copy(ps.at[pl.ds(base, CH)], pv)
    copies = []
    for i in range(NT):
        copies.append(pltpu.async_copy(
            korig.at[pv.at[pl.ds(i * VR, VR)]], kg.at[pl.ds(i * VR, VR)],
            sem))
    kv_back.wait()

    def order_j(j, bad):
        st = jnp.minimum(j * 16, CH - 17)
        a = kv[pl.ds(st, 16)]
        b = kv[pl.ds(st + 1, 16)]
        return bad + jnp.sum(jnp.where((a ^ MSB) <= (b ^ MSB), 0, 1))

    bad = lax.fori_loop(0, VR, order_j, jnp.int32(0))
    for c in copies:
        c.wait()

    def chk_j(j, bad):
        gk = kg[pl.ds(j * 16, 16)]
        kk = kv[pl.ds(j * 16, 16)]
        return bad + jnp.sum(jnp.where(gk == kk, 0, 1))

    bad = lax.fori_loop(0, VR, chk_j, bad)

    @pl.when(t < NT - 1)
    def _():
        pltpu.sync_copy(ks.at[pl.ds((t + 1) * CH, 16)], bw)

    lastv = plsc.load_gather(kv, [zero16 + (CH - 1)])
    bvec = bw[0:16]
    viol = jnp.where((it == 0) & ((lastv ^ MSB) > (bvec ^ MSB)), 1, 0)
    bad = bad + jnp.sum(viol) * jnp.where(t < NT - 1, 1, 0)

    flagmine[0:16] = jnp.where(it == 0, bad, 0)
    pltpu.sync_copy(flagmine, sflag.at[pl.ds(t * 16, 16)])
    plsc.subcore_barrier()

    @pl.when(t == 0)
    def _():
        pltpu.sync_copy(sflag, flagv)

        def red_i(i, acc):
            return acc + jnp.sum(flagv[pl.ds(i * 16, 16)])

        tot_bad = lax.fori_loop(0, NT, red_i, jnp.int32(0))
        okv[0:16] = jnp.where(zero16 + tot_bad == 0, 1, 0)
        pltpu.sync_copy(okv, out_hbm)


_sc_sort = functools.partial(
    pl.kernel,
    out_type=jax.ShapeDtypeStruct((16,), jnp.int32),
    mesh=plsc.VectorSubcoreMesh(
        core_axis_name="c", subcore_axis_name="s", num_cores=1),
    compiler_params=pltpu.CompilerParams(needs_layout_passes=False),
    scratch_types=[
        pltpu.VMEM((CH,), jnp.int32),
        pltpu.VMEM((CH,), jnp.int32),
        pltpu.VMEM((CH,), jnp.int32),
        pltpu.VMEM((NT, VR), jnp.int32),
        pltpu.VMEM((CH,), jnp.int32),
        pltpu.VMEM((CH,), jnp.int32),
        pltpu.VMEM((CH,), jnp.int32),
        pltpu.VMEM((B,), jnp.int32),
        pltpu.VMEM((B,), jnp.int32),
        pltpu.VMEM((NT * B,), jnp.int32),
        pltpu.VMEM((B,), jnp.int32),
        pltpu.VMEM((16,), jnp.int32),
        pltpu.VMEM((NT * 16,), jnp.int32),
        pltpu.VMEM((2, 128), jnp.int32),
        pltpu.VMEM((16,), jnp.int32),
        pltpu.VMEM((16,), jnp.int32),
        pltpu.VMEM((NT * 16,), jnp.int32),
        pltpu.VMEM((16,), jnp.int32),
        pltpu.SemaphoreType.DMA,
        pltpu.SemaphoreType.DMA,
        pltpu.SemaphoreType.DMA,
        pltpu.VMEM_SHARED((N,), jnp.int32),
        pltpu.VMEM_SHARED((N,), jnp.int32),
        pltpu.VMEM_SHARED((N,), jnp.int32),
        pltpu.VMEM_SHARED((N,), jnp.int32),
        pltpu.VMEM_SHARED((N,), jnp.int32),
        pltpu.VMEM_SHARED((NT * B,), jnp.int32),
        pltpu.VMEM_SHARED((NT * B,), jnp.int32),
        pltpu.VMEM_SHARED((NT * 16,), jnp.int32),
        pltpu.VMEM_SHARED((NT * 16,), jnp.int32),
    ],
)(_sc_body)


def kernel(x):
    out = _sc_sort(lax.bitcast_convert_type(x, jnp.int32))
    return out[0].astype(jnp.bool_)


# speedup vs baseline: 1.0002x; 1.0002x over previous
import functools

import jax
import jax.numpy as jnp
import numpy as np
from jax import lax
from jax.experimental import pallas as pl
from jax.experimental.pallas import tpu as pltpu
from jax.experimental.pallas import tpu_sc as plsc

N = 32768
NT = 16
CH = N // NT
VR = CH // 16
RB = 8
B = 1 << RB
NG = B // 16
NPASS = 4
MSB = np.int32(-2147483648)


def _iota16():
    return lax.iota(jnp.int32, 16)


def _sc_body(u_hbm, out_hbm, kv, pv, kg, posv, dv, occv, lv, histv, offv,
             totv, bw, flagmine, flagv, okv, sem, semp, semk,
             sk0, sp0, sk1, sp1, korig, stot, sflag):
    t = lax.axis_index("s")
    it = _iota16()
    base = t * CH
    zero16 = jnp.zeros((16,), jnp.int32)

    pltpu.sync_copy(u_hbm.at[pl.ds(base, CH)], kv)

    def key_j(j, _):
        u = kv[pl.ds(j * 16, 16)]
        s = lax.shift_right_arithmetic(u, 31)
        kv[pl.ds(j * 16, 16)] = (u ^ ~s) & (s | jnp.int32(0x7FFFFFFF))
        pv[pl.ds(j * 16, 16)] = base + j * 16 + it
        return 0

    lax.fori_loop(0, VR, key_j, 0)
    korig_in = pltpu.async_copy(kv, korig.at[pl.ds(base, CH)], semk)

    bufs = [(sk0, sp0), (sk1, sp1)]
    pv_in = None
    for p in range(NPASS):
        shift = RB * p
        k_out, p_out = bufs[p % 2]
        kv_a = kv_b = None
        if p > 0:
            k_in, p_in = bufs[(p + 1) % 2]
            pv_in = pltpu.async_copy(p_in.at[pl.ds(base, CH)], pv, semp)
            H = CH // 2
            kv_a = pltpu.async_copy(
                k_in.at[pl.ds(base, H)], kv.at[pl.ds(0, H)], semk)
            kv_b = pltpu.async_copy(
                k_in.at[pl.ds(base + H, H)], kv.at[pl.ds(H, H)], semk)

        for g in range(NG):
            histv[pl.ds(g * 16, 16)] = zero16
        if kv_a is not None:
            kv_a.wait()

        def hist_j(j, _):
            for u in range(4):
                r = j * 4 + u
                kvec = kv[pl.ds(r * 16, 16)]
                d = lax.shift_right_logical(kvec, shift) & (B - 1)
                occ, last = plsc.scan_count(d)
                dv[pl.ds(r * 16, 16)] = d
                occv[pl.ds(r * 16, 16)] = occ
                lv[pl.ds(r * 16, 16)] = jnp.where(last, 1, 0)
                plsc.addupdate_scatter(histv, [d], occ, mask=last)
            return 0

        lax.fori_loop(0, VR // 8, hist_j, 0)
        if kv_b is not None:
            kv_b.wait()
        lax.fori_loop(VR // 8, VR // 4, hist_j, 0)

        pltpu.sync_copy(histv, stot.at[pl.ds(t * B, B)])
        plsc.subcore_barrier()
        pltpu.sync_copy(stot, totv)

        def scan_g(g, carry):
            tot_g = zero16
            prev_g = zero16
            for tp in range(NT):
                row = totv[pl.ds(tp * B + g * 16, 16)]
                tot_g = tot_g + row
                prev_g = prev_g + jnp.where(
                    lax.full((16,), tp, jnp.int32) < t, row, 0)
            base_g = plsc.cumsum(tot_g) - tot_g + carry
            offv[pl.ds(g * 16, 16)] = base_g + prev_g
            return carry + jnp.sum(tot_g)

        lax.fori_loop(0, NG, scan_g, jnp.int32(0))

        if pv_in is not None:
            pv_in.wait()

        def perm_j(j, _):
            for u in range(2):
                r = j * 2 + u
                d = dv[pl.ds(r * 16, 16)]
                occ = occv[pl.ds(r * 16, 16)]
                last = lv[pl.ds(r * 16, 16)] != 0
                pos = plsc.load_gather(offv, [d]) + occ - 1
                rd = lax.div(r, 8)
                rm = r - rd * 8
                plsc.store_scatter(posv, [zero16 + rd, rm * 16 + it], pos)
                plsc.addupdate_scatter(offv, [d], occ, mask=last)
                if u == 1:
                    @pl.when(rm == 7)
                    def _():
                        pltpu.async_copy(
                            kv.at[pl.ds(rd * VR, VR)],
                            k_out.at[posv.at[rd]], sem)
                        pltpu.async_copy(
                            pv.at[pl.ds(rd * VR, VR)],
                            p_out.at[posv.at[rd]], sem)
            return 0

        lax.fori_loop(0, VR // 2, perm_j, 0)

        for _ in range(2 * NT):
            pltpu.make_async_copy(
                u_hbm.at[pl.ds(0, VR)], kg.at[pl.ds(0, VR)], sem).wait()
        if p == 0:
            korig_in.wait()
        plsc.subcore_barrier()

    ks, ps = bufs[(NPASS - 1) % 2]
    kv_back = pltpu.async_copy(ks.at[pl.ds(base, CH)], kv, semk)
    pltpu.sync_copy(ps.at[pl.ds(base, CH)], pv)
    copies = []
    for i in range(NT):
        copies.append(pltpu.async_copy(
            korig.at[pv.at[pl.ds(i * VR, VR)]], kg.at[pl.ds(i * VR, VR)],
            sem))
    kv_back.wait()

    def order_j(j, bad):
        st = jnp.minimum(j * 16, CH - 17)
        a = kv[pl.ds(st, 16)]
        b = kv[pl.ds(st + 1, 16)]
        return bad + jnp.sum(jnp.where((a ^ MSB) <= (b ^ MSB), 0, 1))

    bad = lax.fori_loop(0, VR, order_j, jnp.int32(0))
    for c in copies:
        c.wait()

    def chk_j(j, bad):
        gk = kg[pl.ds(j * 16, 16)]
        kk = kv[pl.ds(j * 16, 16)]
        return bad + jnp.sum(jnp.where(gk == kk, 0, 1))

    bad = lax.fori_loop(0, VR, chk_j, bad)

    @pl.when(t < NT - 1)
    def _():
        pltpu.sync_copy(ks.at[pl.ds((t + 1) * CH, 16)], bw)

    lastv = plsc.load_gather(kv, [zero16 + (CH - 1)])
    bvec = bw[0:16]
    viol = jnp.where((it == 0) & ((lastv ^ MSB) > (bvec ^ MSB)), 1, 0)
    bad = bad + jnp.sum(viol) * jnp.where(t < NT - 1, 1, 0)

    flagmine[0:16] = jnp.where(it == 0, bad, 0)
    pltpu.sync_copy(flagmine, sflag.at[pl.ds(t * 16, 16)])
    plsc.subcore_barrier()

    @pl.when(t == 0)
    def _():
        pltpu.sync_copy(sflag, flagv)

        def red_i(i, acc):
            return acc + jnp.sum(flagv[pl.ds(i * 16, 16)])

        tot_bad = lax.fori_loop(0, NT, red_i, jnp.int32(0))
        okv[0:16] = jnp.where(zero16 + tot_bad == 0, 1, 0)
        pltpu.sync_copy(okv, out_hbm)


_sc_sort = functools.partial(
    pl.kernel,
    out_type=jax.ShapeDtypeStruct((16,), jnp.int32),
    mesh=plsc.VectorSubcoreMesh(
        core_axis_name="c", subcore_axis_name="s", num_cores=1),
    compiler_params=pltpu.CompilerParams(needs_layout_passes=False),
    scratch_types=[
        pltpu.VMEM((CH,), jnp.int32),
        pltpu.VMEM((CH,), jnp.int32),
        pltpu.VMEM((CH,), jnp.int32),
        pltpu.VMEM((NT, VR), jnp.int32),
        pltpu.VMEM((CH,), jnp.int32),
        pltpu.VMEM((CH,), jnp.int32),
        pltpu.VMEM((CH,), jnp.int32),
        pltpu.VMEM((B,), jnp.int32),
        pltpu.VMEM((B,), jnp.int32),
        pltpu.VMEM((NT * B,), jnp.int32),
        pltpu.VMEM((16,), jnp.int32),
        pltpu.VMEM((16,), jnp.int32),
        pltpu.VMEM((NT * 16,), jnp.int32),
        pltpu.VMEM((16,), jnp.int32),
        pltpu.SemaphoreType.DMA,
        pltpu.SemaphoreType.DMA,
        pltpu.SemaphoreType.DMA,
        pltpu.VMEM_SHARED((N,), jnp.int32),
        pltpu.VMEM_SHARED((N,), jnp.int32),
        pltpu.VMEM_SHARED((N,), jnp.int32),
        pltpu.VMEM_SHARED((N,), jnp.int32),
        pltpu.VMEM_SHARED((N,), jnp.int32),
        pltpu.VMEM_SHARED((NT * B,), jnp.int32),
        pltpu.VMEM_SHARED((NT * 16,), jnp.int32),
    ],
)(_sc_body)


def kernel(x):
    out = _sc_sort(lax.bitcast_convert_type(x, jnp.int32))
    return out[0].astype(jnp.bool_)


# speedup vs baseline: 1.0207x; 1.0205x over previous
import functools

import jax
import jax.numpy as jnp
import numpy as np
from jax import lax
from jax.experimental import pallas as pl
from jax.experimental.pallas import tpu as pltpu
from jax.experimental.pallas import tpu_sc as plsc

N = 32768
NT = 16
CH = N // NT
VR = CH // 16
RB = 8
B = 1 << RB
NG = B // 16
NPASS = 4
MSB = np.int32(-2147483648)


def _iota16():
    return lax.iota(jnp.int32, 16)


def _sc_body(u_hbm, out_hbm, kv, pv, kg, posv, dv, occv, lv, histv, offv,
             totv, bw, flagmine, flagv, okv, sem, semp, semk,
             sk0, sp0, sk1, sp1, korig, stot, sflag):
    t = lax.axis_index("s")
    it = _iota16()
    base = t * CH
    zero16 = jnp.zeros((16,), jnp.int32)

    pltpu.sync_copy(u_hbm.at[pl.ds(base, CH)], kv)

    def key_j(j, _):
        u = kv[pl.ds(j * 16, 16)]
        s = lax.shift_right_arithmetic(u, 31)
        kv[pl.ds(j * 16, 16)] = (u ^ ~s) & (s | jnp.int32(0x7FFFFFFF))
        pv[pl.ds(j * 16, 16)] = base + j * 16 + it
        return 0

    lax.fori_loop(0, VR, key_j, 0)
    korig_in = pltpu.async_copy(kv, korig.at[pl.ds(base, CH)], semk)

    bufs = [(sk0, sp0), (sk1, sp1)]
    pv_in = None
    for p in range(NPASS):
        shift = RB * p
        k_out, p_out = bufs[p % 2]
        kv_a = kv_b = None
        if p > 0:
            k_in, p_in = bufs[(p + 1) % 2]
            pv_in = pltpu.async_copy(p_in.at[pl.ds(base, CH)], pv, semp)
            H = CH // 2
            kv_a = pltpu.async_copy(
                k_in.at[pl.ds(base, H)], kv.at[pl.ds(0, H)], semk)
            kv_b = pltpu.async_copy(
                k_in.at[pl.ds(base + H, H)], kv.at[pl.ds(H, H)], semk)

        for g in range(NG):
            histv[pl.ds(g * 16, 16)] = zero16
        if kv_a is not None:
            kv_a.wait()

        def hist_j(j, _):
            kvec = kv[pl.ds(j * 16, 16)]
            d = lax.shift_right_logical(kvec, shift) & (B - 1)
            occ, last = plsc.scan_count(d)
            dv[pl.ds(j * 16, 16)] = d
            occv[pl.ds(j * 16, 16)] = occ
            lv[pl.ds(j * 16, 16)] = jnp.where(last, 1, 0)
            plsc.addupdate_scatter(histv, [d], occ, mask=last)
            return 0

        lax.fori_loop(0, VR // 2, hist_j, 0)
        if kv_b is not None:
            kv_b.wait()
        lax.fori_loop(VR // 2, VR, hist_j, 0)

        pltpu.sync_copy(histv, stot.at[pl.ds(t * B, B)])
        plsc.subcore_barrier()
        pltpu.sync_copy(stot, totv)

        def scan_g(g, carry):
            tot_g = zero16
            prev_g = zero16
            for tp in range(NT):
                row = totv[pl.ds(tp * B + g * 16, 16)]
                tot_g = tot_g + row
                prev_g = prev_g + jnp.where(
                    lax.full((16,), tp, jnp.int32) < t, row, 0)
            base_g = plsc.cumsum(tot_g) - tot_g + carry
            offv[pl.ds(g * 16, 16)] = base_g + prev_g
            return carry + jnp.sum(tot_g)

        lax.fori_loop(0, NG, scan_g, jnp.int32(0))

        if pv_in is not None:
            pv_in.wait()

        def perm_j(j, _):
            d = dv[pl.ds(j * 16, 16)]
            occ = occv[pl.ds(j * 16, 16)]
            last = lv[pl.ds(j * 16, 16)] != 0
            pos = plsc.load_gather(offv, [d]) + occ - 1
            jd = lax.div(j, 8)
            jm = j - jd * 8
            plsc.store_scatter(posv, [zero16 + jd, jm * 16 + it], pos)
            plsc.addupdate_scatter(offv, [d], occ, mask=last)

            @pl.when(jm == 7)
            def _():
                pltpu.async_copy(
                    kv.at[pl.ds(jd * VR, VR)], k_out.at[posv.at[jd]], sem)
                pltpu.async_copy(
                    pv.at[pl.ds(jd * VR, VR)], p_out.at[posv.at[jd]], sem)

            return 0

        lax.fori_loop(0, VR, perm_j, 0)

        for _ in range(2 * NT):
            pltpu.make_async_copy(
                u_hbm.at[pl.ds(0, VR)], kg.at[pl.ds(0, VR)], sem).wait()
        if p == 0:
            korig_in.wait()
        plsc.subcore_barrier()

    ks, ps = bufs[(NPASS - 1) % 2]
    kv_back = pltpu.async_copy(ks.at[pl.ds(base, CH)], kv, semk)
    pltpu.sync_copy(ps.at[pl.ds(base, CH)], pv)
    copies = []
    for i in range(NT):
        copies.append(pltpu.async_copy(
            korig.at[pv.at[pl.ds(i * VR, VR)]], kg.at[pl.ds(i * VR, VR)],
            sem))
    kv_back.wait()

    def order_j(j, bad):
        st = jnp.minimum(j * 16, CH - 17)
        a = kv[pl.ds(st, 16)]
        b = kv[pl.ds(st + 1, 16)]
        return bad + jnp.sum(jnp.where((a ^ MSB) <= (b ^ MSB), 0, 1))

    bad = lax.fori_loop(0, VR, order_j, jnp.int32(0))
    for c in copies:
        c.wait()

    def chk_j(j, bad):
        gk = kg[pl.ds(j * 16, 16)]
        kk = kv[pl.ds(j * 16, 16)]
        return bad + jnp.sum(jnp.where(gk == kk, 0, 1))

    bad = lax.fori_loop(0, VR, chk_j, bad)

    @pl.when(t < NT - 1)
    def _():
        pltpu.sync_copy(ks.at[pl.ds((t + 1) * CH, 16)], bw)

    lastv = plsc.load_gather(kv, [zero16 + (CH - 1)])
    bvec = bw[0:16]
    viol = jnp.where((it == 0) & ((lastv ^ MSB) > (bvec ^ MSB)), 1, 0)
    bad = bad + jnp.sum(viol) * jnp.where(t < NT - 1, 1, 0)

    flagmine[0:16] = jnp.where(it == 0, bad, 0)
    pltpu.sync_copy(flagmine, sflag.at[pl.ds(t * 16, 16)])
    plsc.subcore_barrier()

    @pl.when(t == 0)
    def _():
        pltpu.sync_copy(sflag, flagv)

        def red_i(i, acc):
            return acc + jnp.sum(flagv[pl.ds(i * 16, 16)])

        tot_bad = lax.fori_loop(0, NT, red_i, jnp.int32(0))
        okv[0:16] = jnp.where(zero16 + tot_bad == 0, 1, 0)
        pltpu.sync_copy(okv, out_hbm)


_sc_sort = functools.partial(
    pl.kernel,
    out_type=jax.ShapeDtypeStruct((16,), jnp.int32),
    mesh=plsc.VectorSubcoreMesh(
        core_axis_name="c", subcore_axis_name="s", num_cores=1),
    compiler_params=pltpu.CompilerParams(needs_layout_passes=False),
    scratch_types=[
        pltpu.VMEM((CH,), jnp.int32),
        pltpu.VMEM((CH,), jnp.int32),
        pltpu.VMEM((CH,), jnp.int32),
        pltpu.VMEM((NT, VR), jnp.int32),
        pltpu.VMEM((CH,), jnp.int32),
        pltpu.VMEM((CH,), jnp.int32),
        pltpu.VMEM((CH,), jnp.int32),
        pltpu.VMEM((B,), jnp.int32),
        pltpu.VMEM((B,), jnp.int32),
        pltpu.VMEM((NT * B,), jnp.int32),
        pltpu.VMEM((16,), jnp.int32),
        pltpu.VMEM((16,), jnp.int32),
        pltpu.VMEM((NT * 16,), jnp.int32),
        pltpu.VMEM((16,), jnp.int32),
        pltpu.SemaphoreType.DMA,
        pltpu.SemaphoreType.DMA,
        pltpu.SemaphoreType.DMA,
        pltpu.VMEM_SHARED((N,), jnp.int32),
        pltpu.VMEM_SHARED((N,), jnp.int32),
        pltpu.VMEM_SHARED((N,), jnp.int32),
        pltpu.VMEM_SHARED((N,), jnp.int32),
        pltpu.VMEM_SHARED((N,), jnp.int32),
        pltpu.VMEM_SHARED((NT * B,), jnp.int32),
        pltpu.VMEM_SHARED((NT * 16,), jnp.int32),
    ],
)(_sc_body)


def kernel(x):
    out = _sc_sort(lax.bitcast_convert_type(x, jnp.int32))
    return out[0].astype(jnp.bool_)
